# Initial kernel scaffold; baseline (speedup 1.0000x reference)
import jax
import jax.numpy as jnp
from jax.experimental import pallas as pl


def kernel(x, edge_index, batch, W1, b1, W2, b2, W3, b3, W4, b4, fcW1, fcb1, fcW2, fcb2):
    raise NotImplementedError("write your pallas kernel here")


# speedup vs baseline: 2.1633x; 2.1633x over previous
import functools

import jax
import jax.numpy as jnp
from jax.experimental import pallas as pl
from jax.experimental.pallas import tpu as pltpu

N = 10000
G = 64
R = 2000


def _mm_first_body(x_ref, deg_ref, W_ref, y_ref, dis_ref):
    dis = jax.lax.rsqrt(deg_ref[...])
    y = jnp.dot(x_ref[...], W_ref[...], preferred_element_type=jnp.float32)
    y_ref[...] = y * dis
    dis_ref[...] = dis


def _mm_mid_body(z_ref, y_prev_ref, dis_ref, b_ref, W_ref, y_ref):
    dis = dis_ref[...]
    h = jnp.maximum(dis * (z_ref[...] + y_prev_ref[...]) + b_ref[...], 0.0)
    y = jnp.dot(h, W_ref[...], preferred_element_type=jnp.float32)
    y_ref[...] = y * dis


def _tail_body(z_ref, y_prev_ref, dis_ref, b_ref, batch_ref, fcW1_ref,
               fcb1_ref, fcW2_ref, fcb2_ref, out_ref, sums_ref, counts_ref):
    i = pl.program_id(0)

    @pl.when(i == 0)
    def _init():
        sums_ref[...] = jnp.zeros_like(sums_ref)
        counts_ref[...] = jnp.zeros_like(counts_ref)

    h = jnp.maximum(dis_ref[...] * (z_ref[...] + y_prev_ref[...]) + b_ref[...], 0.0)
    gids = jax.lax.broadcasted_iota(jnp.int32, (R, G), 1)
    mask = (batch_ref[...] == gids).astype(jnp.float32)
    dn = (((0,), (0,)), ((), ()))
    sums_ref[...] += jax.lax.dot_general(mask, h, dn,
                                         preferred_element_type=jnp.float32)
    counts_ref[...] += jax.lax.dot_general(mask, jnp.ones((R, 1), jnp.float32), dn,
                                           preferred_element_type=jnp.float32)

    @pl.when(i == N // R - 1)
    def _final():
        pooled = sums_ref[...] / jnp.maximum(counts_ref[...], 1.0)
        a1 = jnp.maximum(jnp.dot(pooled, fcW1_ref[...],
                                 preferred_element_type=jnp.float32) + fcb1_ref[...], 0.0)
        o = jnp.dot(a1, fcW2_ref[...], preferred_element_type=jnp.float32) + fcb2_ref[...]
        m = jnp.max(o, axis=1, keepdims=True)
        e = jnp.exp(o - m)
        s = jnp.sum(e, axis=1, keepdims=True)
        out_ref[...] = o - m - jnp.log(s)


def _row_bs(shape):
    return pl.BlockSpec(shape, lambda i: (i, 0))


def _full_bs(shape):
    return pl.BlockSpec(shape, lambda i: (0, 0))


def _mm_first(x, deg, W):
    f_in, f_out = W.shape
    return pl.pallas_call(
        _mm_first_body,
        grid=(N // R,),
        in_specs=[_row_bs((R, f_in)), _row_bs((R, 1)), _full_bs((f_in, f_out))],
        out_specs=[_row_bs((R, f_out)), _row_bs((R, 1))],
        out_shape=[jax.ShapeDtypeStruct((N, f_out), jnp.float32),
                   jax.ShapeDtypeStruct((N, 1), jnp.float32)],
    )(x, deg, W)


def _mm_mid(z, y_prev, dis, b, W):
    f_in, f_out = W.shape
    return pl.pallas_call(
        _mm_mid_body,
        grid=(N // R,),
        in_specs=[_row_bs((R, f_in)), _row_bs((R, f_in)), _row_bs((R, 1)),
                  _full_bs((1, f_in)), _full_bs((f_in, f_out))],
        out_specs=_row_bs((R, f_out)),
        out_shape=jax.ShapeDtypeStruct((N, f_out), jnp.float32),
    )(z, y_prev, dis, b, W)


def _tail(z, y_prev, dis, b, batch2d, fcW1, fcb1, fcW2, fcb2):
    dim = z.shape[1]
    ncls = fcW2.shape[1]
    return pl.pallas_call(
        _tail_body,
        grid=(N // R,),
        in_specs=[_row_bs((R, dim)), _row_bs((R, dim)), _row_bs((R, 1)),
                  _full_bs((1, dim)), _row_bs((R, 1)),
                  _full_bs((dim, dim)), _full_bs((1, dim)),
                  _full_bs((dim, ncls)), _full_bs((1, ncls))],
        out_specs=_full_bs((G, ncls)),
        out_shape=jax.ShapeDtypeStruct((G, ncls), jnp.float32),
        scratch_shapes=[pltpu.VMEM((G, dim), jnp.float32),
                        pltpu.VMEM((G, 1), jnp.float32)],
    )(z, y_prev, dis, b, batch2d, fcW1, fcb1, fcW2, fcb2)


def _aggregate(y, src, dst):
    msgs = y[src]
    return jnp.zeros_like(y).at[dst].add(msgs)


def kernel(x, edge_index, batch, W1, b1, W2, b2, W3, b3, W4, b4,
           fcW1, fcb1, fcW2, fcb2):
    src = edge_index[0].astype(jnp.int32)
    dst = edge_index[1].astype(jnp.int32)
    batch2d = batch.astype(jnp.int32).reshape(N, 1)

    deg = jnp.ones((N,), jnp.float32).at[dst].add(1.0).reshape(N, 1)

    y1, dis = _mm_first(x, deg, W1)
    z1 = _aggregate(y1, src, dst)
    y2 = _mm_mid(z1, y1, dis, b1.reshape(1, -1), W2)
    z2 = _aggregate(y2, src, dst)
    y3 = _mm_mid(z2, y2, dis, b2.reshape(1, -1), W3)
    z3 = _aggregate(y3, src, dst)
    y4 = _mm_mid(z3, y3, dis, b3.reshape(1, -1), W4)
    z4 = _aggregate(y4, src, dst)
    return _tail(z4, y4, dis, b4.reshape(1, -1), batch2d,
                 fcW1, fcb1.reshape(1, -1), fcW2, fcb2.reshape(1, -1))


# speedup vs baseline: 5.3829x; 2.4883x over previous
import functools

import jax
import jax.numpy as jnp
from jax import lax
from jax.experimental import pallas as pl
from jax.experimental.pallas import tpu as pltpu
from jax.experimental.pallas import tpu_sc as plsc

N = 10000
G = 64
R = 2000
RM = 2048
NPAD = 10240
E = 160000
EPAD = 163840
EBLK = 128
ROWS_PER_TILE = EPAD // EBLK // 16
INIT_PER_TILE = NPAD // 16
CH = 128
IDXH = 40


def _row_bs(shape):
    return pl.BlockSpec(shape, lambda i: (i, 0))


def _full_bs(shape):
    return pl.BlockSpec(shape, lambda i: (0, 0))


def _mm_first_body(x_ref, deg_ref, W_ref, y0, y1, y2, y3, dis_ref):
    dis = lax.rsqrt(deg_ref[...] + 1.0)
    y = jnp.dot(x_ref[...], W_ref[...], preferred_element_type=jnp.float32) * dis
    for q, o in enumerate((y0, y1, y2, y3)):
        o[...] = y[:, q * CH:(q + 1) * CH]
    dis_ref[...] = dis


def _mm_first(x, deg, W):
    f_in, f_out = W.shape
    outs = pl.pallas_call(
        _mm_first_body,
        grid=(NPAD // RM,),
        in_specs=[_row_bs((RM, f_in)), _row_bs((RM, 1)), _full_bs((f_in, f_out))],
        out_specs=[_row_bs((RM, CH))] * 4 + [_row_bs((RM, 1))],
        out_shape=[jax.ShapeDtypeStruct((NPAD, CH), jnp.float32)] * 4
        + [jax.ShapeDtypeStruct((NPAD, 1), jnp.float32)],
    )(x, deg, W)
    return outs[:4], outs[4]


def _mm_mid_body(z0, z1, z2, z3, dis_ref, b_ref, W_ref, y0, y1, y2, y3):
    dis = dis_ref[...]
    z = jnp.concatenate([z0[...], z1[...], z2[...], z3[...]], axis=1)
    h = jnp.maximum(dis * z + b_ref[...], 0.0)
    y = jnp.dot(h, W_ref[...], preferred_element_type=jnp.float32) * dis
    for q, o in enumerate((y0, y1, y2, y3)):
        o[...] = y[:, q * CH:(q + 1) * CH]


def _mm_mid(zs, dis, b, W):
    f_in, f_out = W.shape
    return pl.pallas_call(
        _mm_mid_body,
        grid=(NPAD // RM,),
        in_specs=[_row_bs((RM, CH))] * 4
        + [_row_bs((RM, 1)), _full_bs((1, f_in)), _full_bs((f_in, f_out))],
        out_specs=[_row_bs((RM, CH))] * 4,
        out_shape=[jax.ShapeDtypeStruct((NPAD, CH), jnp.float32)] * 4,
    )(*zs, dis, b, W)


def _tail_body(z0, z1, z2, z3, dis_ref, b_ref, batch_ref, fcW1_ref,
               fcb1_ref, fcW2_ref, fcb2_ref, out_ref, sums_ref, counts_ref):
    i = pl.program_id(0)

    @pl.when(i == 0)
    def _init():
        sums_ref[...] = jnp.zeros_like(sums_ref)
        counts_ref[...] = jnp.zeros_like(counts_ref)

    z = jnp.concatenate([z0[...], z1[...], z2[...], z3[...]], axis=1)
    h = jnp.maximum(dis_ref[...] * z + b_ref[...], 0.0)
    gids = lax.broadcasted_iota(jnp.int32, (R, G), 1)
    mask = (batch_ref[...] == gids).astype(jnp.float32)
    dn = (((0,), (0,)), ((), ()))
    sums_ref[...] += lax.dot_general(mask, h, dn,
                                     preferred_element_type=jnp.float32)
    counts_ref[...] += lax.dot_general(mask, jnp.ones((R, 1), jnp.float32), dn,
                                       preferred_element_type=jnp.float32)

    @pl.when(i == N // R - 1)
    def _final():
        pooled = sums_ref[...] / jnp.maximum(counts_ref[...], 1.0)
        a1 = jnp.maximum(jnp.dot(pooled, fcW1_ref[...],
                                 preferred_element_type=jnp.float32) + fcb1_ref[...], 0.0)
        o = jnp.dot(a1, fcW2_ref[...], preferred_element_type=jnp.float32) + fcb2_ref[...]
        m = jnp.max(o, axis=1, keepdims=True)
        e = jnp.exp(o - m)
        s = jnp.sum(e, axis=1, keepdims=True)
        out_ref[...] = o - m - jnp.log(s)


def _tail(zs, dis, b, batch2d, fcW1, fcb1, fcW2, fcb2):
    dim = fcW1.shape[0]
    ncls = fcW2.shape[1]
    return pl.pallas_call(
        _tail_body,
        grid=(N // R,),
        in_specs=[_row_bs((R, CH))] * 4
        + [_row_bs((R, 1)), _full_bs((1, dim)), _row_bs((R, 1)),
           _full_bs((dim, dim)), _full_bs((1, dim)),
           _full_bs((dim, ncls)), _full_bs((1, ncls))],
        out_specs=_full_bs((G, ncls)),
        out_shape=jax.ShapeDtypeStruct((G, ncls), jnp.float32),
        scratch_shapes=[pltpu.VMEM((G, dim), jnp.float32),
                        pltpu.VMEM((G, 1), jnp.float32)],
    )(*zs, dis, b, batch2d, fcW1, fcb1, fcW2, fcb2)


_MESH = dict(core_axis_name="c", subcore_axis_name="s")


def _deg_body(dst2d, zeros_hbm, ones_hbm, deg_out, didx, obuf, acc):
    s = lax.axis_index("s")
    c = lax.axis_index("c")
    pltpu.sync_copy(zeros_hbm, acc.at[pl.ds(640 * s, 640)])
    pltpu.sync_copy(ones_hbm, obuf)
    pltpu.sync_copy(dst2d.at[pl.ds(ROWS_PER_TILE * s, ROWS_PER_TILE)], didx)
    plsc.subcore_barrier()

    def body(b, _):
        pltpu.sync_copy(obuf, acc.at[didx.at[b]], add=True)
        return 0

    lax.fori_loop(0, ROWS_PER_TILE, body, 0)
    plsc.subcore_barrier()

    @pl.when(c == 0)
    def _drain():
        pltpu.sync_copy(acc.at[pl.ds(INIT_PER_TILE * s, INIT_PER_TILE)],
                        deg_out.at[pl.ds(INIT_PER_TILE * s, INIT_PER_TILE)])


def _deg_kernel(dst2d, zeros640, ones128):
    f = pl.kernel(
        _deg_body,
        out_type=jax.ShapeDtypeStruct((NPAD, CH), jnp.float32),
        mesh=plsc.VectorSubcoreMesh(**_MESH),
        scratch_types=[
            pltpu.VMEM((ROWS_PER_TILE, EBLK), jnp.int32),
            pltpu.VMEM((EBLK, CH), jnp.float32),
            pltpu.VMEM_SHARED((NPAD, CH), jnp.float32),
        ],
    )
    return f(dst2d, zeros640, ones128)


def _agg_chunk(yq, zq, src2d, dst2d, sidx, didx, gbuf0, gbuf1, acc,
               gsem0, gsem1, s):
    pltpu.sync_copy(yq.at[pl.ds(INIT_PER_TILE * s, INIT_PER_TILE)],
                    acc.at[pl.ds(INIT_PER_TILE * s, INIT_PER_TILE)])

    plsc.subcore_barrier()

    def pair(i, _):
        b0 = 2 * i
        g0 = pltpu.async_copy(yq.at[sidx.at[b0]], gbuf0, gsem0)
        g1 = pltpu.async_copy(yq.at[sidx.at[b0 + 1]], gbuf1, gsem1)
        g0.wait()
        pltpu.sync_copy(gbuf0, acc.at[didx.at[b0]], add=True)
        g1.wait()
        pltpu.sync_copy(gbuf1, acc.at[didx.at[b0 + 1]], add=True)
        return 0

    for h in range(ROWS_PER_TILE // IDXH):
        pltpu.sync_copy(src2d.at[pl.ds(ROWS_PER_TILE * s + IDXH * h, IDXH)], sidx)
        pltpu.sync_copy(dst2d.at[pl.ds(ROWS_PER_TILE * s + IDXH * h, IDXH)], didx)
        lax.fori_loop(0, IDXH // 2, pair, 0)
    plsc.subcore_barrier()
    pltpu.sync_copy(acc.at[pl.ds(INIT_PER_TILE * s, INIT_PER_TILE)],
                    zq.at[pl.ds(INIT_PER_TILE * s, INIT_PER_TILE)])


def _agg_body(y0, y1, y2, y3, src2d, dst2d, z0, z1, z2, z3,
              sidx, didx, gbuf0, gbuf1, acc, gsem0, gsem1):
    s = lax.axis_index("s")
    c = lax.axis_index("c")

    @pl.when(c == 0)
    def _core0():
        _agg_chunk(y0, z0, src2d, dst2d, sidx, didx, gbuf0, gbuf1, acc,
                   gsem0, gsem1, s)
        _agg_chunk(y1, z1, src2d, dst2d, sidx, didx, gbuf0, gbuf1, acc,
                   gsem0, gsem1, s)

    @pl.when(c == 1)
    def _core1():
        _agg_chunk(y2, z2, src2d, dst2d, sidx, didx, gbuf0, gbuf1, acc,
                   gsem0, gsem1, s)
        _agg_chunk(y3, z3, src2d, dst2d, sidx, didx, gbuf0, gbuf1, acc,
                   gsem0, gsem1, s)


def _aggregate(ys, src2d, dst2d):
    f = pl.kernel(
        _agg_body,
        out_type=[jax.ShapeDtypeStruct((NPAD, CH), jnp.float32)] * 4,
        mesh=plsc.VectorSubcoreMesh(**_MESH),
        scratch_types=[
            pltpu.VMEM((IDXH, EBLK), jnp.int32),
            pltpu.VMEM((IDXH, EBLK), jnp.int32),
            pltpu.VMEM((EBLK, CH), jnp.float32),
            pltpu.VMEM((EBLK, CH), jnp.float32),
            pltpu.VMEM_SHARED((NPAD, CH), jnp.float32),
            pltpu.SemaphoreType.DMA,
            pltpu.SemaphoreType.DMA,
        ],
    )
    return list(f(*ys, src2d, dst2d))


def kernel(x, edge_index, batch, W1, b1, W2, b2, W3, b3, W4, b4,
           fcW1, fcb1, fcW2, fcb2):
    src = edge_index[0].astype(jnp.int32)
    dst = edge_index[1].astype(jnp.int32)
    src2d = jnp.concatenate(
        [src, jnp.zeros((EPAD - E,), jnp.int32)]).reshape(EPAD // EBLK, EBLK)
    dst2d = jnp.concatenate(
        [dst, jnp.full((EPAD - E,), NPAD - 1, jnp.int32)]).reshape(EPAD // EBLK, EBLK)
    batch2d = batch.astype(jnp.int32).reshape(N, 1)
    zeros640 = jnp.zeros((640, CH), jnp.float32)
    ones128 = jnp.ones((EBLK, CH), jnp.float32)

    degraw = _deg_kernel(dst2d, zeros640, ones128)
    deg = degraw[:, :1]

    ys, dis = _mm_first(x, deg, W1)
    zs = _aggregate(ys, src2d, dst2d)
    ys = _mm_mid(zs, dis, b1.reshape(1, -1), W2)
    zs = _aggregate(ys, src2d, dst2d)
    ys = _mm_mid(zs, dis, b2.reshape(1, -1), W3)
    zs = _aggregate(ys, src2d, dst2d)
    ys = _mm_mid(zs, dis, b3.reshape(1, -1), W4)
    zs = _aggregate(ys, src2d, dst2d)
    return _tail(zs, dis, b4.reshape(1, -1), batch2d,
                 fcW1, fcb1.reshape(1, -1), fcW2, fcb2.reshape(1, -1))


# speedup vs baseline: 5.7621x; 1.0704x over previous
import functools

import jax
import jax.numpy as jnp
from jax import lax
from jax.experimental import pallas as pl
from jax.experimental.pallas import tpu as pltpu
from jax.experimental.pallas import tpu_sc as plsc

N = 10000
G = 64
R = 2000
RM = 2048
NPAD = 10240
E = 160000
EPAD = 163840
EBLK = 64
NBUF = 4
ROWS_PER_TILE = EPAD // EBLK // 16
INIT_PER_TILE = NPAD // 16
CH = 128
IDXH = 40


def _row_bs(shape):
    return pl.BlockSpec(shape, lambda i: (i, 0))


def _full_bs(shape):
    return pl.BlockSpec(shape, lambda i: (0, 0))


def _mm_first_body(x_ref, deg_ref, W_ref, y0, y1, y2, y3, dis_ref):
    dis = lax.rsqrt(deg_ref[...] + 1.0)
    y = jnp.dot(x_ref[...], W_ref[...], preferred_element_type=jnp.float32) * dis
    for q, o in enumerate((y0, y1, y2, y3)):
        o[...] = y[:, q * CH:(q + 1) * CH]
    dis_ref[...] = dis


def _mm_first(x, deg, W):
    f_in, f_out = W.shape
    outs = pl.pallas_call(
        _mm_first_body,
        grid=(NPAD // RM,),
        in_specs=[_row_bs((RM, f_in)), _row_bs((RM, 1)), _full_bs((f_in, f_out))],
        out_specs=[_row_bs((RM, CH))] * 4 + [_row_bs((RM, 1))],
        out_shape=[jax.ShapeDtypeStruct((NPAD, CH), jnp.float32)] * 4
        + [jax.ShapeDtypeStruct((NPAD, 1), jnp.float32)],
    )(x, deg, W)
    return outs[:4], outs[4]


def _mm_mid_body(z0, z1, z2, z3, dis_ref, b_ref, W_ref, y0, y1, y2, y3):
    dis = dis_ref[...]
    z = jnp.concatenate([z0[...], z1[...], z2[...], z3[...]], axis=1)
    h = jnp.maximum(dis * z + b_ref[...], 0.0)
    y = jnp.dot(h, W_ref[...], preferred_element_type=jnp.float32) * dis
    for q, o in enumerate((y0, y1, y2, y3)):
        o[...] = y[:, q * CH:(q + 1) * CH]


def _mm_mid(zs, dis, b, W):
    f_in, f_out = W.shape
    return pl.pallas_call(
        _mm_mid_body,
        grid=(NPAD // RM,),
        in_specs=[_row_bs((RM, CH))] * 4
        + [_row_bs((RM, 1)), _full_bs((1, f_in)), _full_bs((f_in, f_out))],
        out_specs=[_row_bs((RM, CH))] * 4,
        out_shape=[jax.ShapeDtypeStruct((NPAD, CH), jnp.float32)] * 4,
    )(*zs, dis, b, W)


def _tail_body(z0, z1, z2, z3, dis_ref, b_ref, batch_ref, fcW1_ref,
               fcb1_ref, fcW2_ref, fcb2_ref, out_ref, sums_ref, counts_ref):
    i = pl.program_id(0)

    @pl.when(i == 0)
    def _init():
        sums_ref[...] = jnp.zeros_like(sums_ref)
        counts_ref[...] = jnp.zeros_like(counts_ref)

    z = jnp.concatenate([z0[...], z1[...], z2[...], z3[...]], axis=1)
    h = jnp.maximum(dis_ref[...] * z + b_ref[...], 0.0)
    gids = lax.broadcasted_iota(jnp.int32, (R, G), 1)
    mask = (batch_ref[...] == gids).astype(jnp.float32)
    dn = (((0,), (0,)), ((), ()))
    sums_ref[...] += lax.dot_general(mask, h, dn,
                                     preferred_element_type=jnp.float32)
    counts_ref[...] += lax.dot_general(mask, jnp.ones((R, 1), jnp.float32), dn,
                                       preferred_element_type=jnp.float32)

    @pl.when(i == N // R - 1)
    def _final():
        pooled = sums_ref[...] / jnp.maximum(counts_ref[...], 1.0)
        a1 = jnp.maximum(jnp.dot(pooled, fcW1_ref[...],
                                 preferred_element_type=jnp.float32) + fcb1_ref[...], 0.0)
        o = jnp.dot(a1, fcW2_ref[...], preferred_element_type=jnp.float32) + fcb2_ref[...]
        m = jnp.max(o, axis=1, keepdims=True)
        e = jnp.exp(o - m)
        s = jnp.sum(e, axis=1, keepdims=True)
        out_ref[...] = o - m - jnp.log(s)


def _tail(zs, dis, b, batch2d, fcW1, fcb1, fcW2, fcb2):
    dim = fcW1.shape[0]
    ncls = fcW2.shape[1]
    return pl.pallas_call(
        _tail_body,
        grid=(N // R,),
        in_specs=[_row_bs((R, CH))] * 4
        + [_row_bs((R, 1)), _full_bs((1, dim)), _row_bs((R, 1)),
           _full_bs((dim, dim)), _full_bs((1, dim)),
           _full_bs((dim, ncls)), _full_bs((1, ncls))],
        out_specs=_full_bs((G, ncls)),
        out_shape=jax.ShapeDtypeStruct((G, ncls), jnp.float32),
        scratch_shapes=[pltpu.VMEM((G, dim), jnp.float32),
                        pltpu.VMEM((G, 1), jnp.float32)],
    )(*zs, dis, b, batch2d, fcW1, fcb1, fcW2, fcb2)


_MESH = dict(core_axis_name="c", subcore_axis_name="s")


def _deg_body(dst2d, zeros_hbm, ones_hbm, deg_out, didx, obuf, acc):
    s = lax.axis_index("s")
    c = lax.axis_index("c")
    pltpu.sync_copy(zeros_hbm, acc.at[pl.ds(640 * s, 640)])
    pltpu.sync_copy(ones_hbm, obuf)
    pltpu.sync_copy(dst2d.at[pl.ds(ROWS_PER_TILE * s, ROWS_PER_TILE)], didx)
    plsc.subcore_barrier()

    def body(b, _):
        pltpu.sync_copy(obuf, acc.at[didx.at[b]], add=True)
        return 0

    lax.fori_loop(0, ROWS_PER_TILE, body, 0)
    plsc.subcore_barrier()

    @pl.when(c == 0)
    def _drain():
        pltpu.sync_copy(acc.at[pl.ds(INIT_PER_TILE * s, INIT_PER_TILE)],
                        deg_out.at[pl.ds(INIT_PER_TILE * s, INIT_PER_TILE)])


def _deg_kernel(dst2d, zeros640, ones128):
    f = pl.kernel(
        _deg_body,
        out_type=jax.ShapeDtypeStruct((NPAD, CH), jnp.float32),
        mesh=plsc.VectorSubcoreMesh(**_MESH),
        scratch_types=[
            pltpu.VMEM((ROWS_PER_TILE, EBLK), jnp.int32),
            pltpu.VMEM((EBLK, CH), jnp.float32),
            pltpu.VMEM_SHARED((NPAD, CH), jnp.float32),
        ],
    )
    return f(dst2d, zeros640, ones128)


def _agg_chunk(yq, zq, src2d, dst2d, sidx, didx, bufs, acc,
               gsems, ssems, s):
    pltpu.sync_copy(yq.at[pl.ds(INIT_PER_TILE * s, INIT_PER_TILE)],
                    acc.at[pl.ds(INIT_PER_TILE * s, INIT_PER_TILE)])

    plsc.subcore_barrier()

    ngrp = IDXH // NBUF

    def group(g, _):
        for j in range(NBUF):
            b = NBUF * g + j
            pltpu.make_async_copy(yq.at[sidx.at[b]], bufs[j], gsems[j]).wait()
            pltpu.async_copy(bufs[j], acc.at[didx.at[b]], ssems[j], add=True)
        for j in range(NBUF):
            @pl.when(g < ngrp - 1)
            def _next():
                b = NBUF * (g + 1) + j
                pltpu.make_async_copy(bufs[j], acc.at[didx.at[b]],
                                      ssems[j]).wait()
                pltpu.async_copy(yq.at[sidx.at[b]], bufs[j], gsems[j])
        return 0

    for h in range(ROWS_PER_TILE // IDXH):
        pltpu.sync_copy(src2d.at[pl.ds(ROWS_PER_TILE * s + IDXH * h, IDXH)], sidx)
        pltpu.sync_copy(dst2d.at[pl.ds(ROWS_PER_TILE * s + IDXH * h, IDXH)], didx)
        for j in range(NBUF):
            pltpu.async_copy(yq.at[sidx.at[j]], bufs[j], gsems[j])
        lax.fori_loop(0, ngrp, group, 0)
        for j in range(NBUF):
            pltpu.make_async_copy(bufs[j], acc.at[didx.at[IDXH - NBUF + j]],
                                  ssems[j]).wait()
    plsc.subcore_barrier()
    pltpu.sync_copy(acc.at[pl.ds(INIT_PER_TILE * s, INIT_PER_TILE)],
                    zq.at[pl.ds(INIT_PER_TILE * s, INIT_PER_TILE)])


def _agg_body(y0, y1, y2, y3, src2d, dst2d, z0, z1, z2, z3,
              sidx, didx, b0, b1, b2, b3, acc,
              gs0, gs1, gs2, gs3, ss0, ss1, ss2, ss3):
    s = lax.axis_index("s")
    c = lax.axis_index("c")
    bufs = (b0, b1, b2, b3)
    gsems = (gs0, gs1, gs2, gs3)
    ssems = (ss0, ss1, ss2, ss3)

    @pl.when(c == 0)
    def _core0():
        _agg_chunk(y0, z0, src2d, dst2d, sidx, didx, bufs, acc, gsems, ssems, s)
        _agg_chunk(y1, z1, src2d, dst2d, sidx, didx, bufs, acc, gsems, ssems, s)

    @pl.when(c == 1)
    def _core1():
        _agg_chunk(y2, z2, src2d, dst2d, sidx, didx, bufs, acc, gsems, ssems, s)
        _agg_chunk(y3, z3, src2d, dst2d, sidx, didx, bufs, acc, gsems, ssems, s)


def _aggregate(ys, src2d, dst2d):
    f = pl.kernel(
        _agg_body,
        out_type=[jax.ShapeDtypeStruct((NPAD, CH), jnp.float32)] * 4,
        mesh=plsc.VectorSubcoreMesh(**_MESH),
        scratch_types=[
            pltpu.VMEM((IDXH, EBLK), jnp.int32),
            pltpu.VMEM((IDXH, EBLK), jnp.int32),
        ] + [pltpu.VMEM((EBLK, CH), jnp.float32)] * NBUF + [
            pltpu.VMEM_SHARED((NPAD, CH), jnp.float32),
        ] + [pltpu.SemaphoreType.DMA] * (2 * NBUF),
    )
    return list(f(*ys, src2d, dst2d))


def kernel(x, edge_index, batch, W1, b1, W2, b2, W3, b3, W4, b4,
           fcW1, fcb1, fcW2, fcb2):
    src = edge_index[0].astype(jnp.int32)
    dst = edge_index[1].astype(jnp.int32)
    src2d = jnp.concatenate(
        [src, jnp.zeros((EPAD - E,), jnp.int32)]).reshape(EPAD // EBLK, EBLK)
    dst2d = jnp.concatenate(
        [dst, jnp.full((EPAD - E,), NPAD - 1, jnp.int32)]).reshape(EPAD // EBLK, EBLK)
    batch2d = batch.astype(jnp.int32).reshape(N, 1)
    zeros640 = jnp.zeros((640, CH), jnp.float32)
    ones128 = jnp.ones((EBLK, CH), jnp.float32)

    degraw = _deg_kernel(dst2d, zeros640, ones128)
    deg = degraw[:, :1]

    ys, dis = _mm_first(x, deg, W1)
    zs = _aggregate(ys, src2d, dst2d)
    ys = _mm_mid(zs, dis, b1.reshape(1, -1), W2)
    zs = _aggregate(ys, src2d, dst2d)
    ys = _mm_mid(zs, dis, b2.reshape(1, -1), W3)
    zs = _aggregate(ys, src2d, dst2d)
    ys = _mm_mid(zs, dis, b3.reshape(1, -1), W4)
    zs = _aggregate(ys, src2d, dst2d)
    return _tail(zs, dis, b4.reshape(1, -1), batch2d,
                 fcW1, fcb1.reshape(1, -1), fcW2, fcb2.reshape(1, -1))


# speedup vs baseline: 5.9427x; 1.0313x over previous
import jax
import jax.numpy as jnp
from jax import lax
from jax.experimental import pallas as pl
from jax.experimental.pallas import tpu as pltpu
from jax.experimental.pallas import tpu_sc as plsc

N = 10000
G = 64
R = 2000
RM = 2048
NPAD = 10240
E = 160000
EPAD = 163840
EBLK = 64
NBUF = 4
ROWS_PER_TILE = EPAD // EBLK // 16
INIT_PER_TILE = NPAD // 16
CH = 128
IDXH = 40


def _row_bs(shape):
    return pl.BlockSpec(shape, lambda i: (i, 0))


def _full_bs(shape):
    return pl.BlockSpec(shape, lambda i: (0, 0))


def _mm_first_body(x_ref, deg_ref, W_ref, y0, y1, y2, y3, dis_ref):
    dis = lax.rsqrt(deg_ref[...] + 1.0)
    y = jnp.dot(x_ref[...], W_ref[...], preferred_element_type=jnp.float32) * dis
    for q, o in enumerate((y0, y1, y2, y3)):
        o[...] = y[:, q * CH:(q + 1) * CH]
    dis_ref[...] = dis


def _mm_first(x, deg, W):
    f_in, f_out = W.shape
    outs = pl.pallas_call(
        _mm_first_body,
        grid=(NPAD // RM,),
        in_specs=[_row_bs((RM, f_in)), _row_bs((RM, 1)), _full_bs((f_in, f_out))],
        out_specs=[_row_bs((RM, CH))] * 4 + [_row_bs((RM, 1))],
        out_shape=[jax.ShapeDtypeStruct((NPAD, CH), jnp.float32)] * 4
        + [jax.ShapeDtypeStruct((NPAD, 1), jnp.float32)],
    )(x, deg, W)
    return outs[:4], outs[4]


def _mm_mid_body(z0, z1, z2, z3, dis_ref, b_ref, W_ref, y0, y1, y2, y3):
    dis = dis_ref[...]
    z = jnp.concatenate([z0[...], z1[...], z2[...], z3[...]], axis=1)
    h = jnp.maximum(dis * z + b_ref[...], 0.0)
    y = jnp.dot(h, W_ref[...], preferred_element_type=jnp.float32) * dis
    for q, o in enumerate((y0, y1, y2, y3)):
        o[...] = y[:, q * CH:(q + 1) * CH]


def _mm_mid(zs, dis, b, W):
    f_in, f_out = W.shape
    return pl.pallas_call(
        _mm_mid_body,
        grid=(NPAD // RM,),
        in_specs=[_row_bs((RM, CH))] * 4
        + [_row_bs((RM, 1)), _full_bs((1, f_in)), _full_bs((f_in, f_out))],
        out_specs=[_row_bs((RM, CH))] * 4,
        out_shape=[jax.ShapeDtypeStruct((NPAD, CH), jnp.float32)] * 4,
    )(*zs, dis, b, W)


def _tail_body(z0, z1, z2, z3, dis_ref, b_ref, batch_ref, fcW1_ref,
               fcb1_ref, fcW2_ref, fcb2_ref, out_ref, sums_ref, counts_ref):
    i = pl.program_id(0)

    @pl.when(i == 0)
    def _init():
        sums_ref[...] = jnp.zeros_like(sums_ref)
        counts_ref[...] = jnp.zeros_like(counts_ref)

    z = jnp.concatenate([z0[...], z1[...], z2[...], z3[...]], axis=1)
    h = jnp.maximum(dis_ref[...] * z + b_ref[...], 0.0)
    gids = lax.broadcasted_iota(jnp.int32, (R, G), 1)
    mask = (batch_ref[...] == gids).astype(jnp.float32)
    dn = (((0,), (0,)), ((), ()))
    sums_ref[...] += lax.dot_general(mask, h, dn,
                                     preferred_element_type=jnp.float32)
    counts_ref[...] += lax.dot_general(mask, jnp.ones((R, 1), jnp.float32), dn,
                                       preferred_element_type=jnp.float32)

    @pl.when(i == N // R - 1)
    def _final():
        pooled = sums_ref[...] / jnp.maximum(counts_ref[...], 1.0)
        a1 = jnp.maximum(jnp.dot(pooled, fcW1_ref[...],
                                 preferred_element_type=jnp.float32) + fcb1_ref[...], 0.0)
        o = jnp.dot(a1, fcW2_ref[...], preferred_element_type=jnp.float32) + fcb2_ref[...]
        m = jnp.max(o, axis=1, keepdims=True)
        e = jnp.exp(o - m)
        s = jnp.sum(e, axis=1, keepdims=True)
        out_ref[...] = o - m - jnp.log(s)


def _tail(zs, dis, b, batch2d, fcW1, fcb1, fcW2, fcb2):
    dim = fcW1.shape[0]
    ncls = fcW2.shape[1]
    return pl.pallas_call(
        _tail_body,
        grid=(N // R,),
        in_specs=[_row_bs((R, CH))] * 4
        + [_row_bs((R, 1)), _full_bs((1, dim)), _row_bs((R, 1)),
           _full_bs((dim, dim)), _full_bs((1, dim)),
           _full_bs((dim, ncls)), _full_bs((1, ncls))],
        out_specs=_full_bs((G, ncls)),
        out_shape=jax.ShapeDtypeStruct((G, ncls), jnp.float32),
        scratch_shapes=[pltpu.VMEM((G, dim), jnp.float32),
                        pltpu.VMEM((G, 1), jnp.float32)],
    )(*zs, dis, b, batch2d, fcW1, fcb1, fcW2, fcb2)


_MESH = dict(core_axis_name="c", subcore_axis_name="s")


def _deg_body(dst2d, zeros_hbm, ones_hbm, deg_out, didx, obuf, acc):
    s = lax.axis_index("s")
    c = lax.axis_index("c")
    pltpu.sync_copy(zeros_hbm, acc.at[pl.ds(640 * s, 640)])
    pltpu.sync_copy(ones_hbm, obuf)
    pltpu.sync_copy(dst2d.at[pl.ds(ROWS_PER_TILE * s, ROWS_PER_TILE)], didx)
    plsc.subcore_barrier()

    def body(b, _):
        pltpu.sync_copy(obuf, acc.at[didx.at[b]], add=True)
        return 0

    lax.fori_loop(0, ROWS_PER_TILE, body, 0)
    plsc.subcore_barrier()

    @pl.when(c == 0)
    def _drain():
        pltpu.sync_copy(acc.at[pl.ds(INIT_PER_TILE * s, INIT_PER_TILE)],
                        deg_out.at[pl.ds(INIT_PER_TILE * s, INIT_PER_TILE)])


def _deg_kernel(dst2d, zeros640, ones128):
    f = pl.kernel(
        _deg_body,
        out_type=jax.ShapeDtypeStruct((NPAD, CH), jnp.float32),
        mesh=plsc.VectorSubcoreMesh(**_MESH),
        scratch_types=[
            pltpu.VMEM((ROWS_PER_TILE, EBLK), jnp.int32),
            pltpu.VMEM((EBLK, CH), jnp.float32),
            pltpu.VMEM_SHARED((NPAD, CH), jnp.float32),
        ],
    )
    return f(dst2d, zeros640, ones128)


def _agg_chunk(yq, zq, src2d, dst2d, sidx, didx, bufs, acc, gsems, ssems, s):
    pltpu.sync_copy(yq.at[pl.ds(INIT_PER_TILE * s, INIT_PER_TILE)],
                    acc.at[pl.ds(INIT_PER_TILE * s, INIT_PER_TILE)])
    plsc.subcore_barrier()

    ngrp = IDXH // 2

    def gather(b, j):
        pltpu.async_copy(yq.at[sidx.at[b]], bufs[j], gsems[j])

    def gather_wait(b, j):
        pltpu.make_async_copy(yq.at[sidx.at[b]], bufs[j], gsems[j]).wait()

    def scatter(b, j):
        pltpu.async_copy(bufs[j], acc.at[didx.at[b]], ssems[j], add=True)

    def scatter_wait(b, j):
        pltpu.make_async_copy(bufs[j], acc.at[didx.at[b]], ssems[j]).wait()

    def pair(t, _):
        for p in range(2):
            g = 2 * t + p
            S = (0, 1) if p == 0 else (2, 3)
            for j in range(2):
                b = 2 * g + j
                gather_wait(b, S[j])
                scatter(b, S[j])
            for j in range(2):
                @pl.when(t < ngrp // 2 - 1)
                def _refill():
                    b = 2 * (g + 2) + j
                    scatter_wait(b, S[j])
                    gather(b, S[j])
        return 0

    for h in range(ROWS_PER_TILE // IDXH):
        pltpu.sync_copy(src2d.at[pl.ds(ROWS_PER_TILE * s + IDXH * h, IDXH)], sidx)
        pltpu.sync_copy(dst2d.at[pl.ds(ROWS_PER_TILE * s + IDXH * h, IDXH)], didx)
        for j in range(NBUF):
            gather(j, j)
        lax.fori_loop(0, ngrp // 2, pair, 0)
        for j in range(NBUF):
            scatter_wait(IDXH - NBUF + j, j)
    plsc.subcore_barrier()
    pltpu.sync_copy(acc.at[pl.ds(INIT_PER_TILE * s, INIT_PER_TILE)],
                    zq.at[pl.ds(INIT_PER_TILE * s, INIT_PER_TILE)])


def _agg_body(y0, y1, y2, y3, src2d, dst2d, z0, z1, z2, z3,
              sidx, didx, b0, b1, b2, b3, acc,
              gs0, gs1, gs2, gs3, ss0, ss1, ss2, ss3):
    s = lax.axis_index("s")
    c = lax.axis_index("c")
    bufs = (b0, b1, b2, b3)
    gsems = (gs0, gs1, gs2, gs3)
    ssems = (ss0, ss1, ss2, ss3)

    @pl.when(c == 0)
    def _core0():
        _agg_chunk(y0, z0, src2d, dst2d, sidx, didx, bufs, acc, gsems, ssems, s)
        _agg_chunk(y1, z1, src2d, dst2d, sidx, didx, bufs, acc, gsems, ssems, s)

    @pl.when(c == 1)
    def _core1():
        _agg_chunk(y2, z2, src2d, dst2d, sidx, didx, bufs, acc, gsems, ssems, s)
        _agg_chunk(y3, z3, src2d, dst2d, sidx, didx, bufs, acc, gsems, ssems, s)


def _aggregate(ys, src2d, dst2d):
    f = pl.kernel(
        _agg_body,
        out_type=[jax.ShapeDtypeStruct((NPAD, CH), jnp.float32)] * 4,
        mesh=plsc.VectorSubcoreMesh(**_MESH),
        scratch_types=[
            pltpu.VMEM((IDXH, EBLK), jnp.int32),
            pltpu.VMEM((IDXH, EBLK), jnp.int32),
        ] + [pltpu.VMEM((EBLK, CH), jnp.float32)] * NBUF + [
            pltpu.VMEM_SHARED((NPAD, CH), jnp.float32),
        ] + [pltpu.SemaphoreType.DMA] * (2 * NBUF),
    )
    return list(f(*ys, src2d, dst2d))


def kernel(x, edge_index, batch, W1, b1, W2, b2, W3, b3, W4, b4,
           fcW1, fcb1, fcW2, fcb2):
    src = edge_index[0].astype(jnp.int32)
    dst = edge_index[1].astype(jnp.int32)
    src2d = jnp.concatenate(
        [src, jnp.zeros((EPAD - E,), jnp.int32)]).reshape(EPAD // EBLK, EBLK)
    dst2d = jnp.concatenate(
        [dst, jnp.full((EPAD - E,), NPAD - 1, jnp.int32)]).reshape(EPAD // EBLK, EBLK)
    batch2d = batch.astype(jnp.int32).reshape(N, 1)
    zeros640 = jnp.zeros((640, CH), jnp.float32)
    ones128 = jnp.ones((EBLK, CH), jnp.float32)

    degraw = _deg_kernel(dst2d, zeros640, ones128)
    deg = degraw[:, :1]

    ys, dis = _mm_first(x, deg, W1)
    zs = _aggregate(ys, src2d, dst2d)
    ys = _mm_mid(zs, dis, b1.reshape(1, -1), W2)
    zs = _aggregate(ys, src2d, dst2d)
    ys = _mm_mid(zs, dis, b2.reshape(1, -1), W3)
    zs = _aggregate(ys, src2d, dst2d)
    ys = _mm_mid(zs, dis, b3.reshape(1, -1), W4)
    zs = _aggregate(ys, src2d, dst2d)
    return _tail(zs, dis, b4.reshape(1, -1), batch2d,
                 fcW1, fcb1.reshape(1, -1), fcW2, fcb2.reshape(1, -1))
